# Initial kernel scaffold; baseline (speedup 1.0000x reference)
import jax
import jax.numpy as jnp
from jax.experimental import pallas as pl


def kernel(xs, W_body, b_body, codebook):
    raise NotImplementedError("write your pallas kernel here")


# speedup vs baseline: 1.0429x; 1.0429x over previous
import functools

import jax
import jax.numpy as jnp
from jax import lax
from jax.experimental import pallas as pl
from jax.experimental.pallas import tpu as pltpu
from jax.experimental.pallas import tpu_sc as plsc

N_EMBED = 8192
EMBED_DIM = 256
FEATURE_DIM = 768
BATCH_TOKENS = 16384

T_BLK = 512
C_BLK = 2048

_NC = 2
_NS = 16
_NW = _NC * _NS
_BPW = BATCH_TOKENS // _NW
_GCHUNK = 128


def _vq_body(xs_ref, wq_ref, b_ref, cbq_ref, cn_ref, h_ref, idx_ref,
             hn_ref, min_ref, arg_ref):
    j = pl.program_id(1)

    @pl.when(j == 0)
    def _init():
        h = lax.dot_general(xs_ref[...], wq_ref[...], (((1,), (0,)), ((), ())),
                            preferred_element_type=jnp.float32) + b_ref[...]
        h_ref[...] = h
        hn_ref[...] = jnp.sum(h * h, axis=1, keepdims=True)
        min_ref[...] = jnp.full(min_ref.shape, jnp.inf, min_ref.dtype)
        arg_ref[...] = jnp.zeros(arg_ref.shape, arg_ref.dtype)

    scores = lax.dot_general(h_ref[...], cbq_ref[...], (((1,), (1,)), ((), ())),
                             preferred_element_type=jnp.float32)
    dist = (hn_ref[...] + cn_ref[...]) - 2.0 * scores
    lmin = jnp.min(dist, axis=1, keepdims=True)
    iota = lax.broadcasted_iota(jnp.int32, dist.shape, 1)
    larg = jnp.min(jnp.where(dist == lmin, iota, jnp.int32(2**30)),
                   axis=1, keepdims=True) + j * C_BLK
    take_new = lmin < min_ref[...]
    min_ref[...] = jnp.where(take_new,
                             lmin.astype(jnp.bfloat16).astype(jnp.float32),
                             min_ref[...])
    arg_ref[...] = jnp.where(take_new, larg, arg_ref[...])

    @pl.when(j == pl.num_programs(1) - 1)
    def _fin():
        idx_ref[...] = arg_ref[...]


_vq_call = pl.pallas_call(
    _vq_body,
    grid=(BATCH_TOKENS // T_BLK, N_EMBED // C_BLK),
    in_specs=[
        pl.BlockSpec((T_BLK, FEATURE_DIM), lambda i, j: (i, 0)),
        pl.BlockSpec((FEATURE_DIM, EMBED_DIM), lambda i, j: (0, 0)),
        pl.BlockSpec((1, EMBED_DIM), lambda i, j: (0, 0)),
        pl.BlockSpec((C_BLK, EMBED_DIM), lambda i, j: (j, 0)),
        pl.BlockSpec((1, C_BLK), lambda i, j: (0, j)),
    ],
    out_specs=[
        pl.BlockSpec((T_BLK, EMBED_DIM), lambda i, j: (i, 0)),
        pl.BlockSpec((T_BLK, 1), lambda i, j: (i, 0)),
    ],
    out_shape=[
        jax.ShapeDtypeStruct((BATCH_TOKENS, EMBED_DIM), jnp.float32),
        jax.ShapeDtypeStruct((BATCH_TOKENS, 1), jnp.int32),
    ],
    scratch_shapes=[
        pltpu.VMEM((T_BLK, 1), jnp.float32),
        pltpu.VMEM((T_BLK, 1), jnp.float32),
        pltpu.VMEM((T_BLK, 1), jnp.int32),
    ],
    compiler_params=pltpu.CompilerParams(
        dimension_semantics=("parallel", "arbitrary"),
    ),
)


@functools.cache
def _make_sc_gather():
    @functools.partial(
        pl.kernel,
        mesh=plsc.VectorSubcoreMesh(core_axis_name="c", subcore_axis_name="s",
                                    num_cores=_NC, num_subcores=_NS),
        out_type=jax.ShapeDtypeStruct((BATCH_TOKENS, EMBED_DIM), jnp.float32),
        scratch_types=[
            pltpu.VMEM((_GCHUNK,), jnp.int32),
            pltpu.VMEM((_GCHUNK, EMBED_DIM), jnp.float32),
            pltpu.SemaphoreType.DMA,
        ],
    )
    def _sc_gather(idx_hbm, table_hbm, out_hbm, idx_v, rows_v, sem):
        wid = lax.axis_index("s") * _NC + lax.axis_index("c")
        base = wid * _BPW
        for g in range(_BPW // _GCHUNK):
            off = base + g * _GCHUNK
            pltpu.sync_copy(idx_hbm.at[pl.ds(off, _GCHUNK)], idx_v)
            pltpu.async_copy(table_hbm.at[idx_v], rows_v, sem).wait()
            pltpu.sync_copy(rows_v, out_hbm.at[pl.ds(off, _GCHUNK)])

    return _sc_gather


def kernel(xs, W_body, b_body, codebook):
    wq = W_body.astype(jnp.bfloat16)
    cbq = codebook.astype(jnp.bfloat16)
    cn = (codebook ** 2).sum(axis=1)
    h, idx = _vq_call(xs, wq, b_body.reshape(1, EMBED_DIM), cbq,
                      cn.reshape(1, N_EMBED))
    rows = _make_sc_gather()(idx.reshape(BATCH_TOKENS), codebook)
    return h + (rows - h)
